# baseline (speedup 1.0000x reference)
import jax
import jax.numpy as jnp
from jax import lax
from jax.experimental import pallas as pl
from jax.experimental.pallas import tpu as pltpu
from jax.experimental.pallas import tpu_sc as plsc

VOCAB = 32
N_D = 128
BATCH = 4096
HIST = 200

NC = 2
NS = 16
NW = NC * NS
N = BATCH * HIST
PER_W = N // NW
G = 128
CHUNK = 256
NG = CHUNK // G
NCHUNK = PER_W // CHUNK


def _emb_body(idx_hbm, table_hbm, out_hbm, idx_v, rows_v, gsem, wsem):
    wid = lax.axis_index("s") * NC + lax.axis_index("c")
    base = wid * PER_W
    pltpu.sync_copy(idx_hbm.at[pl.ds(base, PER_W)], idx_v)

    def fire_gather(i, buf):
        for g in range(NG):
            pltpu.async_copy(
                table_hbm.at[idx_v.at[pl.ds(i * CHUNK + g * G, G)]],
                rows_v.at[buf, pl.ds(g * G, G)],
                gsem,
            )

    def wait_gather(i, buf):
        for g in range(NG):
            pltpu.make_async_copy(
                table_hbm.at[idx_v.at[pl.ds(i * CHUNK + g * G, G)]],
                rows_v.at[buf, pl.ds(g * G, G)],
                gsem,
            ).wait()

    def fire_wb(i, buf):
        pltpu.async_copy(
            rows_v.at[buf], out_hbm.at[pl.ds(base + i * CHUNK, CHUNK)], wsem
        )

    def wait_wb(i, buf):
        pltpu.make_async_copy(
            rows_v.at[buf], out_hbm.at[pl.ds(base + i * CHUNK, CHUNK)], wsem
        ).wait()

    fire_gather(0, 0)
    wait_gather(0, 0)
    fire_gather(1, 1)
    fire_wb(0, 0)

    def chunk_body(i, carry):
        buf = lax.rem(i, 2)
        wait_gather(i, buf)
        wait_wb(i - 1, 1 - buf)
        fire_gather(i + 1, 1 - buf)
        fire_wb(i, buf)
        return carry

    lax.fori_loop(1, NCHUNK - 1, chunk_body, 0)

    last = NCHUNK - 1
    lbuf = last % 2
    wait_gather(last, lbuf)
    wait_wb(last - 1, 1 - lbuf)
    fire_wb(last, lbuf)
    wait_wb(last, lbuf)


@jax.jit
def kernel(input_, weight):
    idx = input_.reshape(N)
    mesh = plsc.VectorSubcoreMesh(core_axis_name="c", subcore_axis_name="s")
    out = pl.kernel(
        _emb_body,
        out_type=jax.ShapeDtypeStruct((N, N_D), jnp.float32),
        mesh=mesh,
        scratch_types=[
            pltpu.VMEM((PER_W,), jnp.int32),
            pltpu.VMEM((2, CHUNK, N_D), jnp.float32),
            pltpu.SemaphoreType.DMA,
            pltpu.SemaphoreType.DMA,
        ],
    )(idx, weight)
    return out.reshape(BATCH, HIST, N_D)


# speedup vs baseline: 11.6822x; 11.6822x over previous
import jax
import jax.numpy as jnp
from jax import lax
from jax.experimental import pallas as pl
from jax.experimental.pallas import tpu as pltpu
from jax.experimental.pallas import tpu_sc as plsc

VOCAB = 32
N_D = 128
BATCH = 4096
HIST = 200

NC = 2
NS = 16
NW = NC * NS
N = BATCH * HIST
PER_W = N // NW
G = 128
CHUNK = 256
NG = CHUNK // G
NCHUNK = PER_W // CHUNK


def _emb_body(idx_hbm, table_hbm, out_hbm, idx_v, table_v, rows_v, gsem, wsem):
    wid = lax.axis_index("s") * NC + lax.axis_index("c")
    base = wid * PER_W

    @pl.when(lax.axis_index("s") == 0)
    def _stage_table():
        pltpu.sync_copy(table_hbm, table_v)

    pltpu.sync_copy(idx_hbm.at[pl.ds(base, PER_W)], idx_v)
    plsc.subcore_barrier()

    def fire_gather(i, buf):
        for g in range(NG):
            pltpu.async_copy(
                table_v.at[idx_v.at[pl.ds(i * CHUNK + g * G, G)]],
                rows_v.at[buf, pl.ds(g * G, G)],
                gsem,
            )

    def wait_gather(i, buf):
        for g in range(NG):
            pltpu.make_async_copy(
                table_v.at[idx_v.at[pl.ds(i * CHUNK + g * G, G)]],
                rows_v.at[buf, pl.ds(g * G, G)],
                gsem,
            ).wait()

    def fire_wb(i, buf):
        pltpu.async_copy(
            rows_v.at[buf], out_hbm.at[pl.ds(base + i * CHUNK, CHUNK)], wsem
        )

    def wait_wb(i, buf):
        pltpu.make_async_copy(
            rows_v.at[buf], out_hbm.at[pl.ds(base + i * CHUNK, CHUNK)], wsem
        ).wait()

    fire_gather(0, 0)
    wait_gather(0, 0)
    fire_gather(1, 1)
    fire_wb(0, 0)

    def chunk_body(i, carry):
        buf = lax.rem(i, 2)
        wait_gather(i, buf)
        wait_wb(i - 1, 1 - buf)
        fire_gather(i + 1, 1 - buf)
        fire_wb(i, buf)
        return carry

    lax.fori_loop(1, NCHUNK - 1, chunk_body, 0)

    last = NCHUNK - 1
    lbuf = last % 2
    wait_gather(last, lbuf)
    wait_wb(last - 1, 1 - lbuf)
    fire_wb(last, lbuf)
    wait_wb(last, lbuf)


@jax.jit
def kernel(input_, weight):
    idx = input_.reshape(N)
    mesh = plsc.VectorSubcoreMesh(core_axis_name="c", subcore_axis_name="s")
    out = pl.kernel(
        _emb_body,
        out_type=jax.ShapeDtypeStruct((N, N_D), jnp.float32),
        mesh=mesh,
        scratch_types=[
            pltpu.VMEM((PER_W,), jnp.int32),
            pltpu.VMEM_SHARED((VOCAB, N_D), jnp.float32),
            pltpu.VMEM((2, CHUNK, N_D), jnp.float32),
            pltpu.SemaphoreType.DMA,
            pltpu.SemaphoreType.DMA,
        ],
    )(idx, weight)
    return out.reshape(BATCH, HIST, N_D)
